# baseline (speedup 1.0000x reference)
import functools

import jax
import jax.numpy as jnp
from jax import lax
from jax.experimental import pallas as pl
from jax.experimental.pallas import tpu as pltpu
from jax.experimental.pallas import tpu_sc as plsc

N = 10000
E = 320000
B = 128
T = 5
LR = 0.01
N_SENSORY = 2048

NC = 2
NSUB = 16
CW = B // NC
CHUNK = 128
NR = 6
NCHUNK = 162
EW = NCHUNK * CHUNK
EPAD = EW * NSUB
SUB_STRIDE = 624
SUB_ROWS = 640

_mesh = plsc.VectorSubcoreMesh(core_axis_name="c", subcore_axis_name="s")


def _sc_pass_body(tab_hbm, gidx_hbm, epk_hbm, out_hbm,
                  gidx_v, est, b0, b1, b2, b3, b4, b5, y_sh, *sems):
    c = lax.axis_index("c")
    s = lax.axis_index("s")
    gsems = sems[0:NR]
    ssems = sems[NR:2 * NR]
    tsems = sems[2 * NR:3 * NR]
    bufs = (b0, b1, b2, b3, b4, b5)
    tab_c = tab_hbm.at[c]

    pltpu.async_copy(gidx_hbm.at[s], gidx_v, gsems[0])
    pltpu.async_copy(epk_hbm.at[s, 0], est.at[0], tsems[0])
    pltpu.async_copy(epk_hbm.at[s, 1], est.at[1], tsems[1])

    def _zrow(j, carry):
        for r in range(CW // 16):
            b0[j, pl.ds(r * 16, 16)] = jnp.zeros((16,), jnp.float32)
        return carry
    lax.fori_loop(0, CHUNK, _zrow, 0)
    base = s * SUB_STRIDE
    for k in range(SUB_ROWS // CHUNK):
        pltpu.async_copy(b0, y_sh.at[pl.ds(base + k * CHUNK, CHUNK)],
                         ssems[0])
    for k in range(SUB_ROWS // CHUNK):
        pltpu.make_async_copy(b0, y_sh.at[pl.ds(base, CHUNK)],
                              ssems[0]).wait()
    pltpu.make_async_copy(gidx_hbm.at[s], gidx_v, gsems[0]).wait()
    plsc.subcore_barrier()

    for k in range(4):
        pltpu.async_copy(tab_c.at[gidx_v.at[k]], bufs[k], gsems[k])

    def _hexa(hi, carry):
        for b in range(NR):
            cc = hi * NR + b
            buf = bufs[b]
            pltpu.make_async_copy(tab_c.at[pl.ds(0, CHUNK)], buf,
                                  gsems[b]).wait()
            pltpu.make_async_copy(epk_hbm.at[s, 0], est.at[b],
                                  tsems[b]).wait()

            def _scale(q, inner):
                for j2 in range(4):
                    wvec = plsc.bitcast(
                        est[b, 1, pl.ds(q * 64 + j2 * 16, 16)], jnp.float32)
                    for l in range(16):
                        wj = wvec[l]
                        e = q * 64 + j2 * 16 + l
                        for r in range(CW // 16):
                            buf[e, pl.ds(r * 16, 16)] = (
                                buf[e, pl.ds(r * 16, 16)] * wj)
                return inner
            lax.fori_loop(0, CHUNK // 64, _scale, 0)

            pltpu.async_copy(buf, y_sh.at[est.at[b, 0]], ssems[b], add=True)

            @pl.when(cc >= 2)
            def _():
                pltpu.make_async_copy(tab_c.at[pl.ds(0, CHUNK)],
                                      bufs[(b + 4) % NR],
                                      ssems[(b + 4) % NR]).wait()

            @pl.when(cc + 2 < NCHUNK)
            def _():
                pltpu.async_copy(epk_hbm.at[s, cc + 2], est.at[(b + 2) % NR],
                                 tsems[(b + 2) % NR])

            @pl.when(cc + 4 < NCHUNK)
            def _():
                pltpu.async_copy(tab_c.at[gidx_v.at[cc + 4]],
                                 bufs[(b + 4) % NR], gsems[(b + 4) % NR])
        return carry
    lax.fori_loop(0, NCHUNK // NR, _hexa, 0)
    for k in (NCHUNK - 2, NCHUNK - 1):
        pltpu.make_async_copy(tab_c.at[pl.ds(0, CHUNK)], bufs[k % NR],
                              ssems[k % NR]).wait()
    plsc.subcore_barrier()

    pltpu.sync_copy(y_sh.at[pl.ds(base, SUB_ROWS)],
                    out_hbm.at[c, pl.ds(base, SUB_ROWS)])


_sc_pass = functools.partial(
    pl.kernel,
    out_type=jax.ShapeDtypeStruct((NC, N, CW), jnp.float32),
    mesh=_mesh,
    scratch_types=[
        pltpu.VMEM((NCHUNK, CHUNK), jnp.int32),
        pltpu.VMEM((NR, 2, CHUNK), jnp.int32),
        pltpu.VMEM((CHUNK, CW), jnp.float32),
        pltpu.VMEM((CHUNK, CW), jnp.float32),
        pltpu.VMEM((CHUNK, CW), jnp.float32),
        pltpu.VMEM((CHUNK, CW), jnp.float32),
        pltpu.VMEM((CHUNK, CW), jnp.float32),
        pltpu.VMEM((CHUNK, CW), jnp.float32),
        pltpu.VMEM_SHARED((N, CW), jnp.float32),
    ] + [pltpu.SemaphoreType.DMA] * (3 * NR),
    compiler_params=pltpu.CompilerParams(use_tc_tiling_on_sc=False,
                                         needs_layout_passes=False),
)(_sc_pass_body)


_RB = 1000
_GRID = N // _RB
_halves = pl.BlockSpec((NC, _RB, CW), lambda i: (0, i, 0))
_full = pl.BlockSpec((_RB, B), lambda i: (i, 0))


def _act_body(v_ref, a_ref):
    v = v_ref[...]
    a_ref[0] = jnp.tanh(v[:, :CW])
    a_ref[1] = jnp.tanh(v[:, CW:])


_act_call = pl.pallas_call(
    _act_body, grid=(_GRID,),
    in_specs=[_full], out_specs=_halves,
    out_shape=jax.ShapeDtypeStruct((NC, N, CW), jnp.float32))


def _err_body(v_ref, p_ref, e_ref):
    v = v_ref[...]
    e_ref[0] = v[:, :CW] - p_ref[0]
    e_ref[1] = v[:, CW:] - p_ref[1]


_err_call = pl.pallas_call(
    _err_body, grid=(_GRID,),
    in_specs=[_full, _halves], out_specs=_halves,
    out_shape=jax.ShapeDtypeStruct((NC, N, CW), jnp.float32))


def _upd_body(v_ref, a_ref, e_ref, b_ref, vo_ref, ao_ref):
    grads = []
    for h in range(NC):
        act = a_ref[h]
        back = b_ref[h] * (1.0 - act * act)
        grads.append(e_ref[h] - back)
    grad = jnp.concatenate(grads, axis=1)
    rows = pl.program_id(0) * _RB + lax.broadcasted_iota(jnp.int32, (_RB, B), 0)
    mask = (rows >= N_SENSORY).astype(jnp.float32)
    vn = v_ref[...] - LR * mask * grad
    vo_ref[...] = vn
    ao_ref[0] = jnp.tanh(vn[:, :CW])
    ao_ref[1] = jnp.tanh(vn[:, CW:])


_upd_call = pl.pallas_call(
    _upd_body, grid=(_GRID,),
    in_specs=[_full, _halves, _halves, _halves],
    out_specs=[_full, _halves],
    out_shape=[jax.ShapeDtypeStruct((N, B), jnp.float32),
               jax.ShapeDtypeStruct((NC, N, CW), jnp.float32)])


def kernel(x, edge_index, weights):
    src = edge_index[0]
    dst = edge_index[1]
    pad = EPAD - E
    srcp = jnp.pad(src, (0, pad)).reshape(NSUB, NCHUNK, CHUNK)
    dstp = jnp.pad(dst, (0, pad)).reshape(NSUB, NCHUNK, CHUNK)
    wbits = jax.lax.bitcast_convert_type(
        jnp.pad(weights, (0, pad)).reshape(NSUB, NCHUNK, CHUNK), jnp.int32)
    epk_fwd = jnp.stack([dstp, wbits], axis=2)
    epk_bwd = jnp.stack([srcp, wbits], axis=2)

    values = x
    act2 = _act_call(values)
    for _ in range(T):
        pred2 = _sc_pass(act2, srcp, epk_fwd)
        err2 = _err_call(values, pred2)
        back2 = _sc_pass(err2, dstp, epk_bwd)
        values, act2 = _upd_call(values, act2, err2, back2)
    return values


# speedup vs baseline: 1.4565x; 1.4565x over previous
import functools

import jax
import jax.numpy as jnp
from jax import lax
from jax.experimental import pallas as pl
from jax.experimental.pallas import tpu as pltpu
from jax.experimental.pallas import tpu_sc as plsc

N = 10000
E = 320000
B = 128
T = 5
LR = 0.01
N_SENSORY = 2048

NC = 2
NSUB = 16
CW = B // NC
CHUNK = 128
NCHUNK = 159
EW = NCHUNK * CHUNK
EPAD = EW * NSUB
SUB_STRIDE = 624
SUB_ROWS = 640

_mesh = plsc.VectorSubcoreMesh(core_axis_name="c", subcore_axis_name="s")


def _sc_pass_body(tab_hbm, gidx_hbm, sidx_hbm, w_hbm, out_hbm,
                  gidx_v, sidx_v, w_v, rows_v0, rows_v1, rows_v2, y_sh,
                  gsem0, gsem1, gsem2, ssem0, ssem1, ssem2):
    c = lax.axis_index("c")
    s = lax.axis_index("s")

    pltpu.async_copy(gidx_hbm.at[s], gidx_v, gsem0)
    pltpu.async_copy(sidx_hbm.at[s], sidx_v, gsem1)
    pltpu.async_copy(w_hbm.at[s], w_v, gsem2)

    def _zrow(j, carry):
        for r in range(CW // 16):
            rows_v0[j, pl.ds(r * 16, 16)] = jnp.zeros((16,), jnp.float32)
        return carry
    lax.fori_loop(0, CHUNK, _zrow, 0)
    base = s * SUB_STRIDE
    for k in range(SUB_ROWS // CHUNK):
        pltpu.async_copy(rows_v0, y_sh.at[pl.ds(base + k * CHUNK, CHUNK)],
                         ssem0)
    for k in range(SUB_ROWS // CHUNK):
        pltpu.make_async_copy(rows_v0, y_sh.at[pl.ds(base, CHUNK)],
                              ssem0).wait()
    pltpu.make_async_copy(gidx_hbm.at[s], gidx_v, gsem0).wait()
    pltpu.make_async_copy(sidx_hbm.at[s], sidx_v, gsem1).wait()
    pltpu.make_async_copy(w_hbm.at[s], w_v, gsem2).wait()
    plsc.subcore_barrier()

    bufs = (rows_v0, rows_v1, rows_v2)
    gsems = (gsem0, gsem1, gsem2)
    ssems = (ssem0, ssem1, ssem2)
    tab_c = tab_hbm.at[c]

    pltpu.async_copy(tab_c.at[gidx_v.at[0]], rows_v0, gsem0)
    pltpu.async_copy(tab_c.at[gidx_v.at[1]], rows_v1, gsem1)

    def _triple(ti, carry):
        for b in range(3):
            cc = ti * 3 + b
            buf = bufs[b]
            pltpu.make_async_copy(tab_c.at[pl.ds(0, CHUNK)], buf,
                                  gsems[b]).wait()

            for j2 in range(CHUNK // 16):
                wvec = w_v[cc, pl.ds(j2 * 16, 16)]
                for l in range(16):
                    wj = wvec[l]
                    e = j2 * 16 + l
                    for r in range(CW // 16):
                        buf[e, pl.ds(r * 16, 16)] = buf[e, pl.ds(r * 16, 16)] * wj

            pltpu.async_copy(buf, y_sh.at[sidx_v.at[cc]], ssems[b], add=True)

            nb = (b + 2) % 3
            nbuf = bufs[nb]

            @pl.when(cc + 2 < NCHUNK)
            def _():
                @pl.when(cc >= 1)
                def _():
                    pltpu.make_async_copy(tab_c.at[pl.ds(0, CHUNK)], nbuf,
                                          ssems[nb]).wait()
                pltpu.async_copy(tab_c.at[gidx_v.at[cc + 2]], nbuf, gsems[nb])
        return carry
    lax.fori_loop(0, NCHUNK // 3, _triple, 0)
    for k in (NCHUNK - 3, NCHUNK - 2, NCHUNK - 1):
        pltpu.make_async_copy(tab_c.at[pl.ds(0, CHUNK)], bufs[k % 3],
                              ssems[k % 3]).wait()
    plsc.subcore_barrier()

    pltpu.sync_copy(y_sh.at[pl.ds(base, SUB_ROWS)],
                    out_hbm.at[c, pl.ds(base, SUB_ROWS)])


_sc_pass = functools.partial(
    pl.kernel,
    out_type=jax.ShapeDtypeStruct((NC, N, CW), jnp.float32),
    mesh=_mesh,
    scratch_types=[
        pltpu.VMEM((NCHUNK, CHUNK), jnp.int32),
        pltpu.VMEM((NCHUNK, CHUNK), jnp.int32),
        pltpu.VMEM((NCHUNK, CHUNK), jnp.float32),
        pltpu.VMEM((CHUNK, CW), jnp.float32),
        pltpu.VMEM((CHUNK, CW), jnp.float32),
        pltpu.VMEM((CHUNK, CW), jnp.float32),
        pltpu.VMEM_SHARED((N, CW), jnp.float32),
        pltpu.SemaphoreType.DMA,
        pltpu.SemaphoreType.DMA,
        pltpu.SemaphoreType.DMA,
        pltpu.SemaphoreType.DMA,
        pltpu.SemaphoreType.DMA,
        pltpu.SemaphoreType.DMA,
    ],
    compiler_params=pltpu.CompilerParams(use_tc_tiling_on_sc=False),
)(_sc_pass_body)


_RB = 1000
_GRID = N // _RB
_halves = pl.BlockSpec((NC, _RB, CW), lambda i: (0, i, 0))
_full = pl.BlockSpec((_RB, B), lambda i: (i, 0))


def _act_body(v_ref, a_ref):
    v = v_ref[...]
    a_ref[0] = jnp.tanh(v[:, :CW])
    a_ref[1] = jnp.tanh(v[:, CW:])


_act_call = pl.pallas_call(
    _act_body, grid=(_GRID,),
    in_specs=[_full], out_specs=_halves,
    out_shape=jax.ShapeDtypeStruct((NC, N, CW), jnp.float32))


def _err_body(v_ref, p_ref, e_ref):
    v = v_ref[...]
    e_ref[0] = v[:, :CW] - p_ref[0]
    e_ref[1] = v[:, CW:] - p_ref[1]


_err_call = pl.pallas_call(
    _err_body, grid=(_GRID,),
    in_specs=[_full, _halves], out_specs=_halves,
    out_shape=jax.ShapeDtypeStruct((NC, N, CW), jnp.float32))


def _upd_body(v_ref, a_ref, e_ref, b_ref, vo_ref, ao_ref):
    grads = []
    for h in range(NC):
        act = a_ref[h]
        back = b_ref[h] * (1.0 - act * act)
        grads.append(e_ref[h] - back)
    grad = jnp.concatenate(grads, axis=1)
    rows = pl.program_id(0) * _RB + lax.broadcasted_iota(jnp.int32, (_RB, B), 0)
    mask = (rows >= N_SENSORY).astype(jnp.float32)
    vn = v_ref[...] - LR * mask * grad
    vo_ref[...] = vn
    ao_ref[0] = jnp.tanh(vn[:, :CW])
    ao_ref[1] = jnp.tanh(vn[:, CW:])


_upd_call = pl.pallas_call(
    _upd_body, grid=(_GRID,),
    in_specs=[_full, _halves, _halves, _halves],
    out_specs=[_full, _halves],
    out_shape=[jax.ShapeDtypeStruct((N, B), jnp.float32),
               jax.ShapeDtypeStruct((NC, N, CW), jnp.float32)])


def kernel(x, edge_index, weights):
    src = edge_index[0]
    dst = edge_index[1]
    pad = EPAD - E
    srcp = jnp.pad(src, (0, pad)).reshape(NSUB, NCHUNK, CHUNK)
    dstp = jnp.pad(dst, (0, pad)).reshape(NSUB, NCHUNK, CHUNK)
    wp = jnp.pad(weights, (0, pad)).reshape(NSUB, NCHUNK, CHUNK)

    values = x
    act2 = _act_call(values)
    for _ in range(T):
        pred2 = _sc_pass(act2, srcp, dstp, wp)
        err2 = _err_call(values, pred2)
        back2 = _sc_pass(err2, dstp, srcp, wp)
        values, act2 = _upd_call(values, act2, err2, back2)
    return values
